# Initial kernel scaffold; baseline (speedup 1.0000x reference)
import jax
import jax.numpy as jnp
from jax.experimental import pallas as pl


def kernel(x, edge_index, query_edges, edge_weight, W1, b1, W2, b2, Wlin, blin):
    raise NotImplementedError("write your pallas kernel here")


# speedup vs baseline: 4.9958x; 4.9958x over previous
import functools

import jax
import jax.numpy as jnp
from jax import lax
from jax.experimental import pallas as pl
from jax.experimental.pallas import tpu as pltpu
from jax.experimental.pallas import tpu_sc as plsc

_NC = 2
_NS = 16
_LANES = 16
_NW = _NC * _NS


def _mm1_body(x_ref, w_ref, o_ref):
    o_ref[...] = jnp.dot(x_ref[...], w_ref[...],
                         preferred_element_type=jnp.float32)


def _mm2_body(a0_ref, a1_ref, b_ref, w_ref, o_ref):
    h = jnp.maximum(a0_ref[...] + a1_ref[...] + b_ref[...], 0.0)
    o_ref[...] = jnp.dot(h, w_ref[...], preferred_element_type=jnp.float32)


def _mm3_body(a0_ref, a1_ref, b_ref, w_ref, bl_ref, o_ref):
    h = a0_ref[...] + a1_ref[...] + b_ref[...]
    o_ref[...] = (jnp.dot(h, w_ref[...], preferred_element_type=jnp.float32)
                  + bl_ref[...])


def _lsm_body(z0_ref, z1_ref, o0_ref, o1_ref):
    z0 = z0_ref[...]
    z1 = z1_ref[...]
    m = jnp.maximum(z0, z1)
    lse = m + jnp.log(jnp.exp(z0 - m) + jnp.exp(z1 - m))
    o0_ref[...] = z0 - lse
    o1_ref[...] = z1 - lse


def _tc_mm1(x, w, blk):
    n, f = x.shape
    h = w.shape[1]
    grid = n // blk
    return pl.pallas_call(
        _mm1_body,
        grid=(grid,),
        in_specs=[
            pl.BlockSpec((blk, f), lambda i: (i, 0)),
            pl.BlockSpec((f, h), lambda i: (0, 0)),
        ],
        out_specs=pl.BlockSpec((blk, h), lambda i: (i, 0)),
        out_shape=jax.ShapeDtypeStruct((n, h), jnp.float32),
    )(x, w)


def _tc_mm2(a0, a1, b_row, w, blk):
    n, f = a0.shape
    h = w.shape[1]
    grid = n // blk
    return pl.pallas_call(
        _mm2_body,
        grid=(grid,),
        in_specs=[
            pl.BlockSpec((blk, f), lambda i: (i, 0)),
            pl.BlockSpec((blk, f), lambda i: (i, 0)),
            pl.BlockSpec((1, f), lambda i: (0, 0)),
            pl.BlockSpec((f, h), lambda i: (0, 0)),
        ],
        out_specs=pl.BlockSpec((blk, h), lambda i: (i, 0)),
        out_shape=jax.ShapeDtypeStruct((n, h), jnp.float32),
    )(a0, a1, b_row, w)


def _tc_mm3(a0, a1, b_row, w8, bl8, blk):
    n, f = a0.shape
    h8 = w8.shape[1]
    grid = n // blk
    return pl.pallas_call(
        _mm3_body,
        grid=(grid,),
        in_specs=[
            pl.BlockSpec((blk, f), lambda i: (i, 0)),
            pl.BlockSpec((blk, f), lambda i: (i, 0)),
            pl.BlockSpec((1, f), lambda i: (0, 0)),
            pl.BlockSpec((f, h8), lambda i: (0, 0)),
            pl.BlockSpec((1, h8), lambda i: (0, 0)),
        ],
        out_specs=pl.BlockSpec((blk, h8), lambda i: (i, 0)),
        out_shape=jax.ShapeDtypeStruct((n, h8), jnp.float32),
    )(a0, a1, b_row, w8, bl8)


def _tc_log_softmax2(z0, z1):
    r, c = z0.shape
    return pl.pallas_call(
        _lsm_body,
        grid=(1,),
        in_specs=[
            pl.BlockSpec((r, c), lambda i: (0, 0)),
            pl.BlockSpec((r, c), lambda i: (0, 0)),
        ],
        out_specs=[
            pl.BlockSpec((r, c), lambda i: (0, 0)),
            pl.BlockSpec((r, c), lambda i: (0, 0)),
        ],
        out_shape=[
            jax.ShapeDtypeStruct((r, c), jnp.float32),
            jax.ShapeDtypeStruct((r, c), jnp.float32),
        ],
    )(z0, z1)


def _make_agg(n, f, ept, ch):
    nch = ept // ch
    dbase = (n // (8 * _NS)) * 8
    tail = n - _NS * dbase
    nzfull, zrem = divmod(dbase, ch)
    assert ch % 8 == 0 and zrem % 8 == 0 and tail % 8 == 0 and tail <= ch
    nvec = f // _LANES
    mesh = plsc.VectorSubcoreMesh(core_axis_name="c", subcore_axis_name="s")

    @functools.partial(
        pl.kernel,
        out_type=jax.ShapeDtypeStruct((_NC, n, f), jnp.float32),
        mesh=mesh,
        compiler_params=pltpu.CompilerParams(needs_layout_passes=False),
        scratch_types=[
            pltpu.VMEM((nch, ch), jnp.int32),
            pltpu.VMEM((nch, ch), jnp.int32),
            pltpu.VMEM((ch,), jnp.float32),
            pltpu.VMEM((ch, f), jnp.float32),
            pltpu.VMEM_SHARED((n, f), jnp.float32),
            pltpu.SemaphoreType.DMA,
        ],
    )
    def agg(h_hbm, src_hbm, dst_hbm, w_hbm, out_hbm,
            src_v, dst_v, w_v, rows_v, acc_sh, sem):
        c = lax.axis_index("c")
        s = lax.axis_index("s")
        wid = c * _NS + s

        def _zfill(i, _):
            for r in range(nvec):
                rows_v[i, pl.ds(r * _LANES, _LANES)] = jnp.zeros(
                    (_LANES,), jnp.float32)
            return 0
        lax.fori_loop(0, ch, _zfill, 0)
        for k in range(nzfull):
            pltpu.sync_copy(rows_v, acc_sh.at[pl.ds(s * dbase + k * ch, ch)])
        if zrem:
            pltpu.sync_copy(rows_v.at[pl.ds(0, zrem)],
                            acc_sh.at[pl.ds(s * dbase + nzfull * ch, zrem)])

        @pl.when(s == 0)
        def _():
            pltpu.sync_copy(rows_v.at[pl.ds(0, tail)],
                            acc_sh.at[pl.ds(_NS * dbase, tail)])
        plsc.subcore_barrier()

        pltpu.sync_copy(src_hbm.at[wid], src_v)
        pltpu.sync_copy(dst_hbm.at[wid], dst_v)

        def _chunk(i, _):
            pltpu.sync_copy(w_hbm.at[pl.ds(wid * ept + i * ch, ch)], w_v)
            pltpu.async_copy(h_hbm.at[src_v.at[i]], rows_v, sem).wait()

            def _scale(j, _):
                wb = plsc.load_gather(
                    w_v, [jnp.full((_LANES,), j, jnp.int32)])
                for r in range(nvec):
                    rows_v[j, pl.ds(r * _LANES, _LANES)] = (
                        rows_v[j, pl.ds(r * _LANES, _LANES)] * wb)
                return 0
            lax.fori_loop(0, ch, _scale, 0)

            pltpu.sync_copy(rows_v, acc_sh.at[dst_v.at[i]], add=True)
            return 0
        lax.fori_loop(0, nch, _chunk, 0)

        plsc.subcore_barrier()

        pltpu.sync_copy(acc_sh.at[pl.ds(s * dbase, dbase)],
                        out_hbm.at[c, pl.ds(s * dbase, dbase)])

        @pl.when(s == 0)
        def _():
            pltpu.sync_copy(acc_sh.at[pl.ds(_NS * dbase, tail)],
                            out_hbm.at[c, pl.ds(_NS * dbase, tail)])

    return agg


def _make_query(n, h8, qpt):
    nq = qpt // _LANES
    mesh = plsc.VectorSubcoreMesh(core_axis_name="c", subcore_axis_name="s")

    @functools.partial(
        pl.kernel,
        out_type=(
            jax.ShapeDtypeStruct((_NW, 1, qpt), jnp.float32),
            jax.ShapeDtypeStruct((_NW, 1, qpt), jnp.float32),
        ),
        mesh=mesh,
        compiler_params=pltpu.CompilerParams(needs_layout_passes=False),
        scratch_types=[
            pltpu.VMEM((n * h8,), jnp.float32),
            pltpu.VMEM((1, qpt), jnp.int32),
            pltpu.VMEM((1, qpt), jnp.int32),
            pltpu.VMEM((1, qpt), jnp.float32),
            pltpu.VMEM((1, qpt), jnp.float32),
        ],
    )
    def qk(pq_hbm, q0_hbm, q1_hbm, z0_hbm, z1_hbm,
           pq_v, q0_v, q1_v, z0_v, z1_v):
        c = lax.axis_index("c")
        s = lax.axis_index("s")
        wid = c * _NS + s
        pltpu.sync_copy(pq_hbm, pq_v)
        pltpu.sync_copy(q0_hbm.at[wid], q0_v)
        pltpu.sync_copy(q1_hbm.at[wid], q1_v)

        def _chunk(i, _):
            q0 = q0_v[0, pl.ds(i * _LANES, _LANES)] * h8
            q1 = q1_v[0, pl.ds(i * _LANES, _LANES)] * h8
            a0 = plsc.load_gather(pq_v, [q0])
            a1 = plsc.load_gather(pq_v, [q0 + 1])
            b0 = plsc.load_gather(pq_v, [q1 + 2])
            b1 = plsc.load_gather(pq_v, [q1 + 3])
            z0_v[0, pl.ds(i * _LANES, _LANES)] = a0 + b0
            z1_v[0, pl.ds(i * _LANES, _LANES)] = a1 + b1
            return 0
        lax.fori_loop(0, nq, _chunk, 0)

        pltpu.sync_copy(z0_v, z0_hbm.at[wid])
        pltpu.sync_copy(z1_v, z1_hbm.at[wid])

    return qk


def kernel(x, edge_index, query_edges, edge_weight, W1, b1, W2, b2, Wlin,
           blin):
    n, f_in = x.shape
    e = edge_index.shape[1]
    q = query_edges.shape[0]
    h = W1.shape[1]
    ept = e // _NW
    ch = 80
    blk = 1000
    h8 = 8

    qpt = -(-q // (_NW * _LANES)) * _LANES
    qpad = _NW * qpt - q

    src3 = edge_index[0].reshape(_NW, ept // ch, ch)
    dst3 = edge_index[1].reshape(_NW, ept // ch, ch)
    qp = jnp.pad(query_edges, ((0, qpad), (0, 0)))
    q0r = qp[:, 0].reshape(_NW, 1, qpt)
    q1r = qp[:, 1].reshape(_NW, 1, qpt)

    w8 = jnp.zeros((h, h8), jnp.float32)
    w8 = w8.at[:, 0:2].set(Wlin[:h])
    w8 = w8.at[:, 2:4].set(Wlin[h:])
    bl8 = jnp.zeros((1, h8), jnp.float32).at[0, 0:2].set(blin)

    agg = _make_agg(n, h, ept, ch)
    qk = _make_query(n, h8, qpt)

    h1 = _tc_mm1(x, W1, blk)
    p1 = agg(h1, src3, dst3, edge_weight)
    h2 = _tc_mm2(p1[0], p1[1], b1.reshape(1, h), W2, blk)
    p2 = agg(h2, src3, dst3, edge_weight)
    pq = _tc_mm3(p2[0], p2[1], b2.reshape(1, h), w8, bl8, blk)
    z0, z1 = qk(pq.reshape(-1), q0r, q1r)
    o0, o1 = _tc_log_softmax2(z0.reshape(_NW, qpt), z1.reshape(_NW, qpt))
    return jnp.stack([o0.reshape(-1)[:q], o1.reshape(-1)[:q]], axis=-1)


# speedup vs baseline: 7.3881x; 1.4789x over previous
import functools

import jax
import jax.numpy as jnp
from jax import lax
from jax.experimental import pallas as pl
from jax.experimental.pallas import tpu as pltpu
from jax.experimental.pallas import tpu_sc as plsc

_NC = 2
_NS = 16
_LANES = 16
_NW = _NC * _NS


def _mm1_body(x_ref, w_ref, o_ref):
    o_ref[...] = jnp.dot(x_ref[...], w_ref[...],
                         preferred_element_type=jnp.float32)


def _mm2_body(a0_ref, a1_ref, b_ref, w_ref, o_ref):
    h = jnp.maximum(a0_ref[...] + a1_ref[...] + b_ref[...], 0.0)
    o_ref[...] = jnp.dot(h, w_ref[...], preferred_element_type=jnp.float32)


def _mm3_body(a0_ref, a1_ref, b_ref, w_ref, bl_ref, o_ref):
    h = a0_ref[...] + a1_ref[...] + b_ref[...]
    o_ref[...] = (jnp.dot(h, w_ref[...], preferred_element_type=jnp.float32)
                  + bl_ref[...])


def _lsm_body(z0_ref, z1_ref, o0_ref, o1_ref):
    z0 = z0_ref[...]
    z1 = z1_ref[...]
    m = jnp.maximum(z0, z1)
    lse = m + jnp.log(jnp.exp(z0 - m) + jnp.exp(z1 - m))
    o0_ref[...] = z0 - lse
    o1_ref[...] = z1 - lse


def _tc_mm1(x, w, blk):
    n, f = x.shape
    h = w.shape[1]
    grid = n // blk
    return pl.pallas_call(
        _mm1_body,
        grid=(grid,),
        in_specs=[
            pl.BlockSpec((blk, f), lambda i: (i, 0)),
            pl.BlockSpec((f, h), lambda i: (0, 0)),
        ],
        out_specs=pl.BlockSpec((blk, h), lambda i: (i, 0)),
        out_shape=jax.ShapeDtypeStruct((n, h), jnp.float32),
    )(x, w)


def _tc_mm2(a0, a1, b_row, w, blk):
    n, f = a0.shape
    h = w.shape[1]
    grid = n // blk
    return pl.pallas_call(
        _mm2_body,
        grid=(grid,),
        in_specs=[
            pl.BlockSpec((blk, f), lambda i: (i, 0)),
            pl.BlockSpec((blk, f), lambda i: (i, 0)),
            pl.BlockSpec((1, f), lambda i: (0, 0)),
            pl.BlockSpec((f, h), lambda i: (0, 0)),
        ],
        out_specs=pl.BlockSpec((blk, h), lambda i: (i, 0)),
        out_shape=jax.ShapeDtypeStruct((n, h), jnp.float32),
    )(a0, a1, b_row, w)


def _tc_mm3(a0, a1, b_row, w8, bl8, blk):
    n, f = a0.shape
    h8 = w8.shape[1]
    grid = n // blk
    return pl.pallas_call(
        _mm3_body,
        grid=(grid,),
        in_specs=[
            pl.BlockSpec((blk, f), lambda i: (i, 0)),
            pl.BlockSpec((blk, f), lambda i: (i, 0)),
            pl.BlockSpec((1, f), lambda i: (0, 0)),
            pl.BlockSpec((f, h8), lambda i: (0, 0)),
            pl.BlockSpec((1, h8), lambda i: (0, 0)),
        ],
        out_specs=pl.BlockSpec((blk, h8), lambda i: (i, 0)),
        out_shape=jax.ShapeDtypeStruct((n, h8), jnp.float32),
    )(a0, a1, b_row, w8, bl8)


def _tc_log_softmax2(z0, z1):
    r, c = z0.shape
    return pl.pallas_call(
        _lsm_body,
        grid=(1,),
        in_specs=[
            pl.BlockSpec((r, c), lambda i: (0, 0)),
            pl.BlockSpec((r, c), lambda i: (0, 0)),
        ],
        out_specs=[
            pl.BlockSpec((r, c), lambda i: (0, 0)),
            pl.BlockSpec((r, c), lambda i: (0, 0)),
        ],
        out_shape=[
            jax.ShapeDtypeStruct((r, c), jnp.float32),
            jax.ShapeDtypeStruct((r, c), jnp.float32),
        ],
    )(z0, z1)


def _make_agg(n, f, ept, ch):
    nch = ept // ch
    npairs = (nch - 1) // 2
    dbase = (n // (8 * _NS)) * 8
    tail = n - _NS * dbase
    nzfull, zrem = divmod(dbase, ch)
    assert ch % _LANES == 0 and zrem % 8 == 0 and tail % 8 == 0 and tail <= ch
    assert nch % 2 == 1
    nvec = f // _LANES
    mesh = plsc.VectorSubcoreMesh(core_axis_name="c", subcore_axis_name="s")

    @functools.partial(
        pl.kernel,
        out_type=jax.ShapeDtypeStruct((_NC, n, f), jnp.float32),
        mesh=mesh,
        compiler_params=pltpu.CompilerParams(needs_layout_passes=False),
        scratch_types=[
            pltpu.VMEM((1, ch), jnp.int32),
            pltpu.VMEM((1, ch), jnp.int32),
            pltpu.VMEM((1, ch), jnp.int32),
            pltpu.VMEM((1, ch), jnp.int32),
            pltpu.VMEM((ch,), jnp.float32),
            pltpu.VMEM((ch,), jnp.float32),
            pltpu.VMEM((ch, f), jnp.float32),
            pltpu.VMEM((ch, f), jnp.float32),
            pltpu.VMEM_SHARED((n, f), jnp.float32),
            pltpu.SemaphoreType.DMA,
            pltpu.SemaphoreType.DMA,
            pltpu.SemaphoreType.DMA,
            pltpu.SemaphoreType.DMA,
        ],
    )
    def agg(h_hbm, src_hbm, dst_hbm, w_hbm, out_hbm,
            src_a, src_b, dst_a, dst_b, w_a, w_b, rows_a, rows_b,
            acc_sh, sem_ia, sem_ib, sem_ga, sem_gb):
        c = lax.axis_index("c")
        s = lax.axis_index("s")
        wid = c * _NS + s

        def _stage(k, src_v, dst_v, w_v, sem):
            pltpu.async_copy(w_hbm.at[pl.ds(wid * ept + k * ch, ch)], w_v,
                             sem)
            pltpu.async_copy(src_hbm.at[wid, k], src_v, sem)
            pltpu.async_copy(dst_hbm.at[wid, k], dst_v, sem)

        def _wait_stage(src_v, dst_v, w_v, sem):
            pltpu.make_async_copy(w_hbm.at[pl.ds(0, ch)], w_v, sem).wait()
            pltpu.make_async_copy(src_hbm.at[wid, 0], src_v, sem).wait()
            pltpu.make_async_copy(dst_hbm.at[wid, 0], dst_v, sem).wait()

        def _gather(src_v, rows_v, sem):
            pltpu.async_copy(h_hbm.at[src_v.at[0]], rows_v, sem)

        def _wait_gather(src_v, rows_v, sem):
            pltpu.make_async_copy(h_hbm.at[src_v.at[0]], rows_v, sem).wait()

        def _scale(w_v, rows_v):
            def _grp(g, _):
                base = g * _LANES
                for j2 in range(_LANES):
                    wb = plsc.load_gather(
                        w_v, [jnp.full((_LANES,), j2, jnp.int32) + base])
                    for r in range(nvec):
                        rows_v[base + j2, pl.ds(r * _LANES, _LANES)] = (
                            rows_v[base + j2, pl.ds(r * _LANES, _LANES)] * wb)
                return 0
            lax.fori_loop(0, ch // _LANES, _grp, 0)

        def _scatter(rows_v, dst_v):
            pltpu.sync_copy(rows_v, acc_sh.at[dst_v.at[0]], add=True)

        def _zfill(i, _):
            for r in range(nvec):
                rows_a[i, pl.ds(r * _LANES, _LANES)] = jnp.zeros(
                    (_LANES,), jnp.float32)
            return 0
        lax.fori_loop(0, ch, _zfill, 0)
        for k in range(nzfull):
            pltpu.sync_copy(rows_a, acc_sh.at[pl.ds(s * dbase + k * ch, ch)])
        if zrem:
            pltpu.sync_copy(rows_a.at[pl.ds(0, zrem)],
                            acc_sh.at[pl.ds(s * dbase + nzfull * ch, zrem)])

        @pl.when(s == 0)
        def _():
            pltpu.sync_copy(rows_a.at[pl.ds(0, tail)],
                            acc_sh.at[pl.ds(_NS * dbase, tail)])

        _stage(0, src_a, dst_a, w_a, sem_ia)
        plsc.subcore_barrier()
        _wait_stage(src_a, dst_a, w_a, sem_ia)
        _gather(src_a, rows_a, sem_ga)

        def _pair(i, _):
            a = 2 * i
            b = a + 1
            _stage(b, src_b, dst_b, w_b, sem_ib)
            _wait_gather(src_a, rows_a, sem_ga)
            _wait_stage(src_b, dst_b, w_b, sem_ib)
            _gather(src_b, rows_b, sem_gb)
            _scale(w_a, rows_a)
            _scatter(rows_a, dst_a)
            _stage(a + 2, src_a, dst_a, w_a, sem_ia)
            _wait_gather(src_b, rows_b, sem_gb)
            _wait_stage(src_a, dst_a, w_a, sem_ia)
            _gather(src_a, rows_a, sem_ga)
            _scale(w_b, rows_b)
            _scatter(rows_b, dst_b)
            return 0
        lax.fori_loop(0, npairs, _pair, 0)

        _wait_gather(src_a, rows_a, sem_ga)
        _scale(w_a, rows_a)
        _scatter(rows_a, dst_a)

        plsc.subcore_barrier()

        pltpu.sync_copy(acc_sh.at[pl.ds(s * dbase, dbase)],
                        out_hbm.at[c, pl.ds(s * dbase, dbase)])

        @pl.when(s == 0)
        def _():
            pltpu.sync_copy(acc_sh.at[pl.ds(_NS * dbase, tail)],
                            out_hbm.at[c, pl.ds(_NS * dbase, tail)])

    return agg


def _make_query(n, h8, qpt):
    nq = qpt // _LANES
    mesh = plsc.VectorSubcoreMesh(core_axis_name="c", subcore_axis_name="s")

    @functools.partial(
        pl.kernel,
        out_type=(
            jax.ShapeDtypeStruct((_NW, 1, qpt), jnp.float32),
            jax.ShapeDtypeStruct((_NW, 1, qpt), jnp.float32),
        ),
        mesh=mesh,
        compiler_params=pltpu.CompilerParams(needs_layout_passes=False),
        scratch_types=[
            pltpu.VMEM((n * h8,), jnp.float32),
            pltpu.VMEM((1, qpt), jnp.int32),
            pltpu.VMEM((1, qpt), jnp.int32),
            pltpu.VMEM((1, qpt), jnp.float32),
            pltpu.VMEM((1, qpt), jnp.float32),
        ],
    )
    def qk(pq_hbm, q0_hbm, q1_hbm, z0_hbm, z1_hbm,
           pq_v, q0_v, q1_v, z0_v, z1_v):
        c = lax.axis_index("c")
        s = lax.axis_index("s")
        wid = c * _NS + s
        pltpu.sync_copy(pq_hbm, pq_v)
        pltpu.sync_copy(q0_hbm.at[wid], q0_v)
        pltpu.sync_copy(q1_hbm.at[wid], q1_v)

        def _chunk(i, _):
            q0 = q0_v[0, pl.ds(i * _LANES, _LANES)] * h8
            q1 = q1_v[0, pl.ds(i * _LANES, _LANES)] * h8
            a0 = plsc.load_gather(pq_v, [q0])
            a1 = plsc.load_gather(pq_v, [q0 + 1])
            b0 = plsc.load_gather(pq_v, [q1 + 2])
            b1 = plsc.load_gather(pq_v, [q1 + 3])
            z0_v[0, pl.ds(i * _LANES, _LANES)] = a0 + b0
            z1_v[0, pl.ds(i * _LANES, _LANES)] = a1 + b1
            return 0
        lax.fori_loop(0, nq, _chunk, 0)

        pltpu.sync_copy(z0_v, z0_hbm.at[wid])
        pltpu.sync_copy(z1_v, z1_hbm.at[wid])

    return qk


def kernel(x, edge_index, query_edges, edge_weight, W1, b1, W2, b2, Wlin,
           blin):
    n, f_in = x.shape
    e = edge_index.shape[1]
    q = query_edges.shape[0]
    h = W1.shape[1]
    ept = e // _NW
    ch = 80
    blk = 1000
    h8 = 8

    qpt = -(-q // (_NW * _LANES)) * _LANES
    qpad = _NW * qpt - q

    src4 = edge_index[0].reshape(_NW, ept // ch, 1, ch)
    dst4 = edge_index[1].reshape(_NW, ept // ch, 1, ch)
    qp = jnp.pad(query_edges, ((0, qpad), (0, 0)))
    q0r = qp[:, 0].reshape(_NW, 1, qpt)
    q1r = qp[:, 1].reshape(_NW, 1, qpt)

    w8 = jnp.zeros((h, h8), jnp.float32)
    w8 = w8.at[:, 0:2].set(Wlin[:h])
    w8 = w8.at[:, 2:4].set(Wlin[h:])
    bl8 = jnp.zeros((1, h8), jnp.float32).at[0, 0:2].set(blin)

    agg = _make_agg(n, h, ept, ch)
    qk = _make_query(n, h8, qpt)

    h1 = _tc_mm1(x, W1, blk)
    p1 = agg(h1, src4, dst4, edge_weight)
    h2 = _tc_mm2(p1[0], p1[1], b1.reshape(1, h), W2, blk)
    p2 = agg(h2, src4, dst4, edge_weight)
    pq = _tc_mm3(p2[0], p2[1], b2.reshape(1, h), w8, bl8, blk)
    z0, z1 = qk(pq.reshape(-1), q0r, q1r)
    o0, o1 = _tc_log_softmax2(z0.reshape(_NW, qpt), z1.reshape(_NW, qpt))
    return jnp.stack([o0.reshape(-1)[:q], o1.reshape(-1)[:q]], axis=-1)


# speedup vs baseline: 9.0169x; 1.2205x over previous
import functools

import jax
import jax.numpy as jnp
from jax import lax
from jax.experimental import pallas as pl
from jax.experimental.pallas import tpu as pltpu
from jax.experimental.pallas import tpu_sc as plsc

_NC = 2
_NS = 16
_LANES = 16
_NW = _NC * _NS


def _mm1_body(x_ref, w_ref, o_ref):
    o_ref[...] = jnp.dot(x_ref[...], w_ref[...],
                         preferred_element_type=jnp.float32)


def _mm2_body(a0_ref, a1_ref, b_ref, w_ref, o_ref):
    h = jnp.maximum(a0_ref[...] + a1_ref[...] + b_ref[...], 0.0)
    o_ref[...] = jnp.dot(h, w_ref[...], preferred_element_type=jnp.float32)


def _mm3_body(a0_ref, a1_ref, b_ref, w_ref, bl_ref, o_ref):
    h = a0_ref[...] + a1_ref[...] + b_ref[...]
    o_ref[...] = (jnp.dot(h, w_ref[...], preferred_element_type=jnp.float32)
                  + bl_ref[...])


def _lsm_body(z0_ref, z1_ref, o0_ref, o1_ref):
    z0 = z0_ref[...]
    z1 = z1_ref[...]
    m = jnp.maximum(z0, z1)
    lse = m + jnp.log(jnp.exp(z0 - m) + jnp.exp(z1 - m))
    o0_ref[...] = z0 - lse
    o1_ref[...] = z1 - lse


def _tc_mm1(x, w, blk):
    n, f = x.shape
    h = w.shape[1]
    grid = n // blk
    return pl.pallas_call(
        _mm1_body,
        grid=(grid,),
        in_specs=[
            pl.BlockSpec((blk, f), lambda i: (i, 0)),
            pl.BlockSpec((f, h), lambda i: (0, 0)),
        ],
        out_specs=pl.BlockSpec((blk, h), lambda i: (i, 0)),
        out_shape=jax.ShapeDtypeStruct((n, h), jnp.float32),
    )(x, w)


def _tc_mm2(a0, a1, b_row, w, blk):
    n, f = a0.shape
    h = w.shape[1]
    grid = n // blk
    return pl.pallas_call(
        _mm2_body,
        grid=(grid,),
        in_specs=[
            pl.BlockSpec((blk, f), lambda i: (i, 0)),
            pl.BlockSpec((blk, f), lambda i: (i, 0)),
            pl.BlockSpec((1, f), lambda i: (0, 0)),
            pl.BlockSpec((f, h), lambda i: (0, 0)),
        ],
        out_specs=pl.BlockSpec((blk, h), lambda i: (i, 0)),
        out_shape=jax.ShapeDtypeStruct((n, h), jnp.float32),
    )(a0, a1, b_row, w)


def _tc_mm3(a0, a1, b_row, w8, bl8, blk):
    n, f = a0.shape
    h8 = w8.shape[1]
    grid = n // blk
    return pl.pallas_call(
        _mm3_body,
        grid=(grid,),
        in_specs=[
            pl.BlockSpec((blk, f), lambda i: (i, 0)),
            pl.BlockSpec((blk, f), lambda i: (i, 0)),
            pl.BlockSpec((1, f), lambda i: (0, 0)),
            pl.BlockSpec((f, h8), lambda i: (0, 0)),
            pl.BlockSpec((1, h8), lambda i: (0, 0)),
        ],
        out_specs=pl.BlockSpec((blk, h8), lambda i: (i, 0)),
        out_shape=jax.ShapeDtypeStruct((n, h8), jnp.float32),
    )(a0, a1, b_row, w8, bl8)


def _tc_log_softmax2(z0, z1):
    r, c = z0.shape
    return pl.pallas_call(
        _lsm_body,
        grid=(1,),
        in_specs=[
            pl.BlockSpec((r, c), lambda i: (0, 0)),
            pl.BlockSpec((r, c), lambda i: (0, 0)),
        ],
        out_specs=[
            pl.BlockSpec((r, c), lambda i: (0, 0)),
            pl.BlockSpec((r, c), lambda i: (0, 0)),
        ],
        out_shape=[
            jax.ShapeDtypeStruct((r, c), jnp.float32),
            jax.ShapeDtypeStruct((r, c), jnp.float32),
        ],
    )(z0, z1)


def _make_agg(n, f, ept, ch):
    nch = ept // ch
    nquad = (nch - 1) // 4
    dbase = (n // (8 * _NS)) * 8
    tail = n - _NS * dbase
    nzfull, zrem = divmod(dbase, ch)
    assert ch % _LANES == 0 and zrem % 8 == 0 and tail % 8 == 0 and tail <= ch
    assert nch == 4 * nquad + 1
    nvec = f // _LANES
    mesh = plsc.VectorSubcoreMesh(core_axis_name="c", subcore_axis_name="s")

    @functools.partial(
        pl.kernel,
        out_type=jax.ShapeDtypeStruct((_NC, n, f), jnp.float32),
        mesh=mesh,
        compiler_params=pltpu.CompilerParams(needs_layout_passes=False),
        scratch_types=[
            [pltpu.VMEM((1, ch), jnp.int32)] * 4,
            [pltpu.VMEM((1, ch), jnp.int32)] * 4,
            [pltpu.VMEM((ch,), jnp.float32)] * 4,
            [pltpu.VMEM((ch, f), jnp.float32)] * 4,
            pltpu.VMEM_SHARED((n, f), jnp.float32),
            [pltpu.SemaphoreType.DMA] * 4,
            [pltpu.SemaphoreType.DMA] * 4,
            [pltpu.SemaphoreType.DMA] * 4,
        ],
    )
    def agg(h_hbm, src_hbm, dst_hbm, w_hbm, out_hbm,
            src_r, dst_r, w_r, rows_r, acc_sh, sem_i, sem_g, sem_s):
        c = lax.axis_index("c")
        s = lax.axis_index("s")
        wid = c * _NS + s

        def _stage(k, u):
            pltpu.async_copy(w_hbm.at[pl.ds(wid * ept + k * ch, ch)],
                             w_r[u], sem_i[u])
            pltpu.async_copy(src_hbm.at[wid, k], src_r[u], sem_i[u])
            pltpu.async_copy(dst_hbm.at[wid, k], dst_r[u], sem_i[u])

        def _wait_stage(u):
            pltpu.make_async_copy(w_hbm.at[pl.ds(0, ch)], w_r[u],
                                  sem_i[u]).wait()
            pltpu.make_async_copy(src_hbm.at[wid, 0], src_r[u],
                                  sem_i[u]).wait()
            pltpu.make_async_copy(dst_hbm.at[wid, 0], dst_r[u],
                                  sem_i[u]).wait()

        def _gather(u):
            pltpu.async_copy(h_hbm.at[src_r[u].at[0]], rows_r[u], sem_g[u])

        def _wait_gather(u):
            pltpu.make_async_copy(h_hbm.at[src_r[u].at[0]], rows_r[u],
                                  sem_g[u]).wait()

        def _scale(u):
            w_v = w_r[u]
            rows_v = rows_r[u]

            def _grp(g, _):
                base = g * _LANES
                for j2 in range(_LANES):
                    wb = plsc.load_gather(
                        w_v, [jnp.full((_LANES,), j2, jnp.int32) + base])
                    for r in range(nvec):
                        rows_v[base + j2, pl.ds(r * _LANES, _LANES)] = (
                            rows_v[base + j2, pl.ds(r * _LANES, _LANES)] * wb)
                return 0
            lax.fori_loop(0, ch // _LANES, _grp, 0)

        def _scatter(u):
            pltpu.async_copy(rows_r[u], acc_sh.at[dst_r[u].at[0]], sem_s[u],
                             add=True)

        def _wait_scatter(u):
            pltpu.make_async_copy(rows_r[u], acc_sh.at[dst_r[u].at[0]],
                                  sem_s[u]).wait()

        def _zfill(i, _):
            for r in range(nvec):
                rows_r[0][i, pl.ds(r * _LANES, _LANES)] = jnp.zeros(
                    (_LANES,), jnp.float32)
            return 0
        lax.fori_loop(0, ch, _zfill, 0)
        for k in range(nzfull):
            pltpu.sync_copy(rows_r[0],
                            acc_sh.at[pl.ds(s * dbase + k * ch, ch)])
        if zrem:
            pltpu.sync_copy(rows_r[0].at[pl.ds(0, zrem)],
                            acc_sh.at[pl.ds(s * dbase + nzfull * ch, zrem)])

        @pl.when(s == 0)
        def _():
            pltpu.sync_copy(rows_r[0].at[pl.ds(0, tail)],
                            acc_sh.at[pl.ds(_NS * dbase, tail)])

        _stage(0, 0)
        _stage(1, 1)
        plsc.subcore_barrier()
        _wait_stage(0)
        _gather(0)
        _wait_stage(1)
        _gather(1)

        def _quad(i, _):
            for kk in range(4):
                k = 4 * i + kk
                u = kk
                v = (kk + 2) % 4
                _wait_gather(u)
                _scale(u)
                _scatter(u)
                if kk < 2:
                    @pl.when(i > 0)
                    def _():
                        _wait_scatter(v)
                else:
                    _wait_scatter(v)
                if kk == 3:
                    @pl.when(i < nquad - 1)
                    def _():
                        _stage(k + 2, v)
                        _wait_stage(v)
                        _gather(v)
                else:
                    _stage(k + 2, v)
                    _wait_stage(v)
                    _gather(v)
            return 0
        lax.fori_loop(0, nquad, _quad, 0)

        u_last = (nch - 1) % 4
        _wait_gather(u_last)
        _scale(u_last)
        _scatter(u_last)
        for k in (nch - 3, nch - 2, nch - 1):
            _wait_scatter(k % 4)

        plsc.subcore_barrier()

        pltpu.sync_copy(acc_sh.at[pl.ds(s * dbase, dbase)],
                        out_hbm.at[c, pl.ds(s * dbase, dbase)])

        @pl.when(s == 0)
        def _():
            pltpu.sync_copy(acc_sh.at[pl.ds(_NS * dbase, tail)],
                            out_hbm.at[c, pl.ds(_NS * dbase, tail)])

    return agg


def _make_query(n, h8, qpt):
    nq = qpt // _LANES
    mesh = plsc.VectorSubcoreMesh(core_axis_name="c", subcore_axis_name="s")

    @functools.partial(
        pl.kernel,
        out_type=(
            jax.ShapeDtypeStruct((_NW, 1, qpt), jnp.float32),
            jax.ShapeDtypeStruct((_NW, 1, qpt), jnp.float32),
        ),
        mesh=mesh,
        compiler_params=pltpu.CompilerParams(needs_layout_passes=False),
        scratch_types=[
            pltpu.VMEM((n * h8,), jnp.float32),
            pltpu.VMEM((1, qpt), jnp.int32),
            pltpu.VMEM((1, qpt), jnp.int32),
            pltpu.VMEM((1, qpt), jnp.float32),
            pltpu.VMEM((1, qpt), jnp.float32),
        ],
    )
    def qk(pq_hbm, q0_hbm, q1_hbm, z0_hbm, z1_hbm,
           pq_v, q0_v, q1_v, z0_v, z1_v):
        c = lax.axis_index("c")
        s = lax.axis_index("s")
        wid = c * _NS + s
        pltpu.sync_copy(pq_hbm, pq_v)
        pltpu.sync_copy(q0_hbm.at[wid], q0_v)
        pltpu.sync_copy(q1_hbm.at[wid], q1_v)

        def _chunk(i, _):
            q0 = q0_v[0, pl.ds(i * _LANES, _LANES)] * h8
            q1 = q1_v[0, pl.ds(i * _LANES, _LANES)] * h8
            a0 = plsc.load_gather(pq_v, [q0])
            a1 = plsc.load_gather(pq_v, [q0 + 1])
            b0 = plsc.load_gather(pq_v, [q1 + 2])
            b1 = plsc.load_gather(pq_v, [q1 + 3])
            z0_v[0, pl.ds(i * _LANES, _LANES)] = a0 + b0
            z1_v[0, pl.ds(i * _LANES, _LANES)] = a1 + b1
            return 0
        lax.fori_loop(0, nq, _chunk, 0)

        pltpu.sync_copy(z0_v, z0_hbm.at[wid])
        pltpu.sync_copy(z1_v, z1_hbm.at[wid])

    return qk


def kernel(x, edge_index, query_edges, edge_weight, W1, b1, W2, b2, Wlin,
           blin):
    n, f_in = x.shape
    e = edge_index.shape[1]
    q = query_edges.shape[0]
    h = W1.shape[1]
    ept = e // _NW
    ch = 80
    blk = 1000
    h8 = 8

    qpt = -(-q // (_NW * _LANES)) * _LANES
    qpad = _NW * qpt - q

    src4 = edge_index[0].reshape(_NW, ept // ch, 1, ch)
    dst4 = edge_index[1].reshape(_NW, ept // ch, 1, ch)
    qp = jnp.pad(query_edges, ((0, qpad), (0, 0)))
    q0r = qp[:, 0].reshape(_NW, 1, qpt)
    q1r = qp[:, 1].reshape(_NW, 1, qpt)

    w8 = jnp.zeros((h, h8), jnp.float32)
    w8 = w8.at[:, 0:2].set(Wlin[:h])
    w8 = w8.at[:, 2:4].set(Wlin[h:])
    bl8 = jnp.zeros((1, h8), jnp.float32).at[0, 0:2].set(blin)

    agg = _make_agg(n, h, ept, ch)
    qk = _make_query(n, h8, qpt)

    h1 = _tc_mm1(x, W1, blk)
    p1 = agg(h1, src4, dst4, edge_weight)
    h2 = _tc_mm2(p1[0], p1[1], b1.reshape(1, h), W2, blk)
    p2 = agg(h2, src4, dst4, edge_weight)
    pq = _tc_mm3(p2[0], p2[1], b2.reshape(1, h), w8, bl8, blk)
    z0, z1 = qk(pq.reshape(-1), q0r, q1r)
    o0, o1 = _tc_log_softmax2(z0.reshape(_NW, qpt), z1.reshape(_NW, qpt))
    return jnp.stack([o0.reshape(-1)[:q], o1.reshape(-1)[:q]], axis=-1)
